# baseline (speedup 1.0000x reference)
import jax
import jax.numpy as jnp
from jax import lax
from jax.experimental import pallas as pl
from jax.experimental.pallas import tpu as pltpu
from jax.experimental.pallas import tpu_sc as plsc

N = 10000
E = 320000
D = 128
HID = 64

NC = 2
NS = 16
NW = NC * NS
EPT = E // NW
B = 40
NB = EPT // B
NBUF = 3
RPT = 624
RLAST = N - 15 * RPT
DW = 8


def _sc_body(x_hbm, src_hbm, dst_hbm, zsum_hbm, zdeg_hbm, ones_hbm,
             sum_out, deg_out,
             src_v, dst_v, rv0, rv1, rv2, ones_v, sum_sh, deg_sh,
             g0, g1, g2, s0, s1, s2, d0):
    c = lax.axis_index("c")
    s = lax.axis_index("s")
    w = c * NS + s
    rvs = [rv0, rv1, rv2]
    gs = [g0, g1, g2]
    ss = [s0, s1, s2]

    @pl.when(s < NS - 1)
    def _():
        pltpu.sync_copy(zsum_hbm.at[pl.ds(0, RPT)],
                        sum_sh.at[pl.ds(s * RPT, RPT)])
        pltpu.sync_copy(zdeg_hbm.at[pl.ds(0, RPT)],
                        deg_sh.at[pl.ds(s * RPT, RPT)])

    @pl.when(s == NS - 1)
    def _():
        pltpu.sync_copy(zsum_hbm, sum_sh.at[pl.ds(N - RLAST, RLAST)])
        pltpu.sync_copy(zdeg_hbm, deg_sh.at[pl.ds(N - RLAST, RLAST)])

    pltpu.sync_copy(ones_hbm, ones_v)
    pltpu.sync_copy(src_hbm.at[w], src_v)
    pltpu.sync_copy(dst_hbm.at[w], dst_v)
    plsc.subcore_barrier()

    def gstart(j, b):
        pltpu.async_copy(x_hbm.at[src_v.at[j]], rvs[b], gs[b])

    def gwait(b):
        pltpu.make_async_copy(x_hbm.at[src_v.at[0]], rvs[b], gs[b]).wait()

    def sstart(j, b):
        pltpu.async_copy(rvs[b], sum_sh.at[dst_v.at[j]], ss[b], add=True)

    def swait(b):
        pltpu.make_async_copy(rvs[b], sum_sh.at[dst_v.at[0]], ss[b]).wait()

    def dstart(j):
        pltpu.async_copy(ones_v, deg_sh.at[dst_v.at[j]], d0, add=True)

    def dwait():
        pltpu.make_async_copy(ones_v, deg_sh.at[dst_v.at[0]], d0).wait()

    for b in range(NBUF - 1):
        gstart(b, b)

    def group(g, carry):
        for b in range(NBUF):
            j = g * NBUF + b
            gwait(b)
            sstart(j, b)
            if b == 0:
                @pl.when(g > 0)
                def _():
                    swait(NBUF - 1)
                    dwait()
            else:
                swait(b - 1)
                dwait()
            dstart(j)
            gstart(jnp.minimum(j + NBUF - 1, NB - 1), (b + NBUF - 1) % NBUF)
        return carry

    NG = (NB - 1) // NBUF
    lax.fori_loop(0, NG, group, 0)

    for j in range(NG * NBUF, NB):
        b = j % NBUF
        gwait(b)
        sstart(j, b)
        swait((j - 1) % NBUF)
        dwait()
        dstart(j)

    swait((NB - 1) % NBUF)
    dwait()
    for k in range((NBUF - 1) + NG * NBUF - NB):
        gwait((NB + k) % NBUF)
    plsc.subcore_barrier()

    @pl.when(s < NS - 1)
    def _():
        pltpu.sync_copy(sum_sh.at[pl.ds(s * RPT, RPT)],
                        sum_out.at[c].at[pl.ds(s * RPT, RPT)])
        pltpu.sync_copy(deg_sh.at[pl.ds(s * RPT, RPT)],
                        deg_out.at[c].at[pl.ds(s * RPT, RPT)])

    @pl.when(s == NS - 1)
    def _():
        pltpu.sync_copy(sum_sh.at[pl.ds(N - RLAST, RLAST)],
                        sum_out.at[c].at[pl.ds(N - RLAST, RLAST)])
        pltpu.sync_copy(deg_sh.at[pl.ds(N - RLAST, RLAST)],
                        deg_out.at[c].at[pl.ds(N - RLAST, RLAST)])


def _sc_segment_sum(x, src_r, dst_r, zsum, zdeg, ones):
    mesh = plsc.VectorSubcoreMesh(core_axis_name="c", subcore_axis_name="s")
    fn = pl.kernel(
        _sc_body,
        out_type=[
            jax.ShapeDtypeStruct((NC, N, D), jnp.float32),
            jax.ShapeDtypeStruct((NC, N, DW), jnp.float32),
        ],
        mesh=mesh,
        scratch_types=[
            pltpu.VMEM((NB, B), jnp.int32),
            pltpu.VMEM((NB, B), jnp.int32),
            pltpu.VMEM((B, D), jnp.float32),
            pltpu.VMEM((B, D), jnp.float32),
            pltpu.VMEM((B, D), jnp.float32),
            pltpu.VMEM((B, DW), jnp.float32),
            pltpu.VMEM_SHARED((N, D), jnp.float32),
            pltpu.VMEM_SHARED((N, DW), jnp.float32),
            pltpu.SemaphoreType.DMA,
            pltpu.SemaphoreType.DMA,
            pltpu.SemaphoreType.DMA,
            pltpu.SemaphoreType.DMA,
            pltpu.SemaphoreType.DMA,
            pltpu.SemaphoreType.DMA,
            pltpu.SemaphoreType.DMA,
        ],
        compiler_params=pltpu.CompilerParams(use_tc_tiling_on_sc=False),
        name="sc_segment_sum",
    )
    return fn(x, src_r, dst_r, zsum, zdeg, ones)


BM = 1000


def _mlp_body(x_ref, sum_ref, deg_ref, w1a_ref, w1b_ref, b1_ref, w2_ref,
              b2_ref, o_ref):
    xb = x_ref[...]
    sb = sum_ref[0] + sum_ref[1]
    dg = deg_ref[0, :, 0:1] + deg_ref[1, :, 0:1]
    mean = jnp.where(dg > 0.0, sb / jnp.maximum(dg, 1.0), xb)
    h = jnp.dot(xb, w1a_ref[...], preferred_element_type=jnp.float32)
    h += jnp.dot(mean, w1b_ref[...], preferred_element_type=jnp.float32)
    h = jnp.maximum(h + b1_ref[...], 0.0)
    o_ref[...] = (jnp.dot(h, w2_ref[...], preferred_element_type=jnp.float32)
                  + b2_ref[...])


def _mlp(x, sum_p, deg_p, w1a, w1b, b1, w2, b2):
    return pl.pallas_call(
        _mlp_body,
        grid=(N // BM,),
        in_specs=[
            pl.BlockSpec((BM, D), lambda i: (i, 0)),
            pl.BlockSpec((NC, BM, D), lambda i: (0, i, 0)),
            pl.BlockSpec((NC, BM, DW), lambda i: (0, i, 0)),
            pl.BlockSpec((D, HID), lambda i: (0, 0)),
            pl.BlockSpec((D, HID), lambda i: (0, 0)),
            pl.BlockSpec((1, HID), lambda i: (0, 0)),
            pl.BlockSpec((HID, D), lambda i: (0, 0)),
            pl.BlockSpec((1, D), lambda i: (0, 0)),
        ],
        out_specs=pl.BlockSpec((BM, D), lambda i: (i, 0)),
        out_shape=jax.ShapeDtypeStruct((N, D), jnp.float32),
        name="mlp_mixer",
    )(x, sum_p, deg_p, w1a, w1b, b1, w2, b2)


def kernel(x, edge_index, W1, b1, W2, b2):
    src_r = edge_index[0].reshape(NW, NB, B)
    dst_r = edge_index[1].reshape(NW, NB, B)
    zsum = jnp.zeros((RLAST, D), jnp.float32)
    zdeg = jnp.zeros((RLAST, DW), jnp.float32)
    ones = jnp.ones((B, DW), jnp.float32)
    sum_p, deg_p = _sc_segment_sum(x, src_r, dst_r, zsum, zdeg, ones)
    return _mlp(x, sum_p, deg_p, W1[:D], W1[D:], b1.reshape(1, HID), W2,
                b2.reshape(1, D))


# speedup vs baseline: 1.2194x; 1.2194x over previous
import jax
import jax.numpy as jnp
from jax import lax
from jax.experimental import pallas as pl
from jax.experimental.pallas import tpu as pltpu
from jax.experimental.pallas import tpu_sc as plsc

N = 10000
E = 320000
D = 128
HID = 64

NC = 2
NS = 16
NW = NC * NS
EPT = E // NW
B = 80
NB = EPT // B
NBUF = 4
NIDX = 8
UNROLL = 8
RPT = 624
RLAST = N - 15 * RPT
DW = 8


def _sc_body(x_hbm, src_hbm, dst_hbm, zsum_hbm, zdeg_hbm, ones_hbm,
             sum_out, deg_out,
             srcx_v, dstx_v, rv0, rv1, rv2, rv3, ones_v, sum_sh, deg_sh,
             g0, g1, g2, g3, s0, s1, s2, s3, d0,
             i0, i1, i2, i3, i4, i5, i6, i7):
    c = lax.axis_index("c")
    s = lax.axis_index("s")
    w = c * NS + s
    rvs = [rv0, rv1, rv2, rv3]
    gs = [g0, g1, g2, g3]
    ss = [s0, s1, s2, s3]
    isem = [i0, i1, i2, i3, i4, i5, i6, i7]

    @pl.when(s < NS - 1)
    def _():
        pltpu.sync_copy(zsum_hbm.at[pl.ds(0, RPT)],
                        sum_sh.at[pl.ds(s * RPT, RPT)])
        pltpu.sync_copy(zdeg_hbm.at[pl.ds(0, RPT)],
                        deg_sh.at[pl.ds(s * RPT, RPT)])

    @pl.when(s == NS - 1)
    def _():
        pltpu.sync_copy(zsum_hbm, sum_sh.at[pl.ds(N - RLAST, RLAST)])
        pltpu.sync_copy(zdeg_hbm, deg_sh.at[pl.ds(N - RLAST, RLAST)])

    pltpu.sync_copy(ones_hbm, ones_v)
    plsc.subcore_barrier()

    def istart(j, slot):
        pltpu.async_copy(src_hbm.at[w].at[j], srcx_v.at[slot], isem[slot])
        pltpu.async_copy(dst_hbm.at[w].at[j], dstx_v.at[slot], isem[slot])

    def iwait(slot):
        pltpu.make_async_copy(src_hbm.at[w].at[0], srcx_v.at[slot],
                              isem[slot]).wait()
        pltpu.make_async_copy(dst_hbm.at[w].at[0], dstx_v.at[slot],
                              isem[slot]).wait()

    def gstart(slot, b):
        pltpu.async_copy(x_hbm.at[srcx_v.at[slot]], rvs[b], gs[b])

    def gwait(b):
        pltpu.make_async_copy(x_hbm.at[srcx_v.at[0]], rvs[b], gs[b]).wait()

    def sstart(slot, b):
        pltpu.async_copy(rvs[b], sum_sh.at[dstx_v.at[slot]], ss[b], add=True)

    def swait(b):
        pltpu.make_async_copy(rvs[b], sum_sh.at[dstx_v.at[0]], ss[b]).wait()

    def dstart(slot):
        pltpu.async_copy(ones_v, deg_sh.at[dstx_v.at[slot]], d0, add=True)

    def dwait():
        pltpu.make_async_copy(ones_v, deg_sh.at[dstx_v.at[0]], d0).wait()

    for t in range(NIDX - 1):
        istart(t, t)
    for t in range(NBUF - 1):
        iwait(t)
        gstart(t, t)

    def group(g, carry):
        for t in range(UNROLL):
            j = g * UNROLL + t
            b = t % NBUF
            gwait(b)
            sstart(t % NIDX, b)
            if t == 0:
                @pl.when(g > 0)
                def _():
                    swait(NBUF - 1)
                    dwait()
            else:
                swait((t - 1) % NBUF)
                dwait()
            dstart(t % NIDX)
            istart(jnp.minimum(j + NIDX - 1, NB - 1), (t + NIDX - 1) % NIDX)
            iwait((t + NBUF - 1) % NIDX)
            gstart((t + NBUF - 1) % NIDX, (b + NBUF - 1) % NBUF)
        return carry

    NG = (NB - (NBUF - 1) - 1) // UNROLL
    lax.fori_loop(0, NG, group, 0)

    for j in range(NG * UNROLL, NB):
        b = j % NBUF
        gwait(b)
        sstart(j % NIDX, b)
        swait((j - 1) % NBUF)
        dwait()
        dstart(j % NIDX)
        if j + NBUF - 1 <= NB - 1:
            iwait((j + NBUF - 1) % NIDX)
            gstart((j + NBUF - 1) % NIDX, (j + NBUF - 1) % NBUF)

    swait((NB - 1) % NBUF)
    dwait()
    for k in range((NIDX - 1) + NG * UNROLL - NB):
        iwait((NB + k) % NIDX)
    plsc.subcore_barrier()

    @pl.when(s < NS - 1)
    def _():
        pltpu.sync_copy(sum_sh.at[pl.ds(s * RPT, RPT)],
                        sum_out.at[c].at[pl.ds(s * RPT, RPT)])
        pltpu.sync_copy(deg_sh.at[pl.ds(s * RPT, RPT)],
                        deg_out.at[c].at[pl.ds(s * RPT, RPT)])

    @pl.when(s == NS - 1)
    def _():
        pltpu.sync_copy(sum_sh.at[pl.ds(N - RLAST, RLAST)],
                        sum_out.at[c].at[pl.ds(N - RLAST, RLAST)])
        pltpu.sync_copy(deg_sh.at[pl.ds(N - RLAST, RLAST)],
                        deg_out.at[c].at[pl.ds(N - RLAST, RLAST)])


def _sc_segment_sum(x, src_r, dst_r, zsum, zdeg, ones):
    mesh = plsc.VectorSubcoreMesh(core_axis_name="c", subcore_axis_name="s")
    fn = pl.kernel(
        _sc_body,
        out_type=[
            jax.ShapeDtypeStruct((NC, N, D), jnp.float32),
            jax.ShapeDtypeStruct((NC, N, DW), jnp.float32),
        ],
        mesh=mesh,
        scratch_types=[
            pltpu.VMEM((NIDX, B), jnp.int32),
            pltpu.VMEM((NIDX, B), jnp.int32),
            pltpu.VMEM((B, D), jnp.float32),
            pltpu.VMEM((B, D), jnp.float32),
            pltpu.VMEM((B, D), jnp.float32),
            pltpu.VMEM((B, D), jnp.float32),
            pltpu.VMEM((B, DW), jnp.float32),
            pltpu.VMEM_SHARED((N, D), jnp.float32),
            pltpu.VMEM_SHARED((N, DW), jnp.float32),
            pltpu.SemaphoreType.DMA,
            pltpu.SemaphoreType.DMA,
            pltpu.SemaphoreType.DMA,
            pltpu.SemaphoreType.DMA,
            pltpu.SemaphoreType.DMA,
            pltpu.SemaphoreType.DMA,
            pltpu.SemaphoreType.DMA,
            pltpu.SemaphoreType.DMA,
            pltpu.SemaphoreType.DMA,
            pltpu.SemaphoreType.DMA,
            pltpu.SemaphoreType.DMA,
            pltpu.SemaphoreType.DMA,
            pltpu.SemaphoreType.DMA,
            pltpu.SemaphoreType.DMA,
            pltpu.SemaphoreType.DMA,
            pltpu.SemaphoreType.DMA,
            pltpu.SemaphoreType.DMA,
        ],
        compiler_params=pltpu.CompilerParams(use_tc_tiling_on_sc=False),
        name="sc_segment_sum",
    )
    return fn(x, src_r, dst_r, zsum, zdeg, ones)


BM = 1000


def _mlp_body(x_ref, sum_ref, deg_ref, w1a_ref, w1b_ref, b1_ref, w2_ref,
              b2_ref, o_ref):
    xb = x_ref[...]
    sb = sum_ref[0] + sum_ref[1]
    dg = deg_ref[0, :, 0:1] + deg_ref[1, :, 0:1]
    mean = jnp.where(dg > 0.0, sb / jnp.maximum(dg, 1.0), xb)
    h = jnp.dot(xb, w1a_ref[...], preferred_element_type=jnp.float32)
    h += jnp.dot(mean, w1b_ref[...], preferred_element_type=jnp.float32)
    h = jnp.maximum(h + b1_ref[...], 0.0)
    o_ref[...] = (jnp.dot(h, w2_ref[...], preferred_element_type=jnp.float32)
                  + b2_ref[...])


def _mlp(x, sum_p, deg_p, w1a, w1b, b1, w2, b2):
    return pl.pallas_call(
        _mlp_body,
        grid=(N // BM,),
        in_specs=[
            pl.BlockSpec((BM, D), lambda i: (i, 0)),
            pl.BlockSpec((NC, BM, D), lambda i: (0, i, 0)),
            pl.BlockSpec((NC, BM, DW), lambda i: (0, i, 0)),
            pl.BlockSpec((D, HID), lambda i: (0, 0)),
            pl.BlockSpec((D, HID), lambda i: (0, 0)),
            pl.BlockSpec((1, HID), lambda i: (0, 0)),
            pl.BlockSpec((HID, D), lambda i: (0, 0)),
            pl.BlockSpec((1, D), lambda i: (0, 0)),
        ],
        out_specs=pl.BlockSpec((BM, D), lambda i: (i, 0)),
        out_shape=jax.ShapeDtypeStruct((N, D), jnp.float32),
        name="mlp_mixer",
    )(x, sum_p, deg_p, w1a, w1b, b1, w2, b2)


def kernel(x, edge_index, W1, b1, W2, b2):
    src_r = edge_index[0].reshape(NW, NB, B)
    dst_r = edge_index[1].reshape(NW, NB, B)
    zsum = jnp.zeros((RLAST, D), jnp.float32)
    zdeg = jnp.zeros((RLAST, DW), jnp.float32)
    ones = jnp.ones((B, DW), jnp.float32)
    sum_p, deg_p = _sc_segment_sum(x, src_r, dst_r, zsum, zdeg, ones)
    return _mlp(x, sum_p, deg_p, W1[:D], W1[D:], b1.reshape(1, HID), W2,
                b2.reshape(1, D))


# speedup vs baseline: 1.2238x; 1.0036x over previous
import jax
import jax.numpy as jnp
from jax import lax
from jax.experimental import pallas as pl
from jax.experimental.pallas import tpu as pltpu
from jax.experimental.pallas import tpu_sc as plsc

N = 10000
E = 320000
D = 128
HID = 64

NC = 2
NS = 16
NW = NC * NS
EPT = E // NW
B = 80
NB = EPT // B
NBUF = 4
NIDX = 8
UNROLL = 8
RPT = 624
RLAST = N - 15 * RPT
DW = 8


def _sc_body(x_hbm, src_hbm, dst_hbm, zsum_hbm, zdeg_hbm, ones_hbm,
             sum_out, deg_out,
             srcx_v, dstx_v, rv0, rv1, rv2, rv3, ones_v, sum_sh, deg_sh,
             g0, g1, g2, g3, s0, s1, s2, s3, d0,
             i0, i1, i2, i3, i4, i5, i6, i7):
    c = lax.axis_index("c")
    s = lax.axis_index("s")
    w = c * NS + s
    rvs = [rv0, rv1, rv2, rv3]
    gs = [g0, g1, g2, g3]
    ss = [s0, s1, s2, s3]
    isem = [i0, i1, i2, i3, i4, i5, i6, i7]

    @pl.when(s < NS - 1)
    def _():
        pltpu.sync_copy(zsum_hbm.at[pl.ds(0, RPT)],
                        sum_sh.at[pl.ds(s * RPT, RPT)])
        pltpu.sync_copy(zdeg_hbm.at[pl.ds(0, RPT)],
                        deg_sh.at[pl.ds(s * RPT, RPT)])

    @pl.when(s == NS - 1)
    def _():
        pltpu.sync_copy(zsum_hbm, sum_sh.at[pl.ds(N - RLAST, RLAST)])
        pltpu.sync_copy(zdeg_hbm, deg_sh.at[pl.ds(N - RLAST, RLAST)])

    pltpu.sync_copy(ones_hbm, ones_v)

    def istart(j, slot):
        pltpu.async_copy(src_hbm.at[w].at[j], srcx_v.at[slot], isem[slot])
        pltpu.async_copy(dst_hbm.at[w].at[j], dstx_v.at[slot], isem[slot])

    def iwait(slot):
        pltpu.make_async_copy(src_hbm.at[w].at[0], srcx_v.at[slot],
                              isem[slot]).wait()
        pltpu.make_async_copy(dst_hbm.at[w].at[0], dstx_v.at[slot],
                              isem[slot]).wait()

    def gstart(slot, b):
        pltpu.async_copy(x_hbm.at[srcx_v.at[slot]], rvs[b], gs[b])

    def gwait(b):
        pltpu.make_async_copy(x_hbm.at[srcx_v.at[0]], rvs[b], gs[b]).wait()

    def sstart(slot, b):
        pltpu.async_copy(rvs[b], sum_sh.at[dstx_v.at[slot]], ss[b], add=True)

    def swait(b):
        pltpu.make_async_copy(rvs[b], sum_sh.at[dstx_v.at[0]], ss[b]).wait()

    def dstart(slot):
        pltpu.async_copy(ones_v, deg_sh.at[dstx_v.at[slot]], d0, add=True)

    def dwait():
        pltpu.make_async_copy(ones_v, deg_sh.at[dstx_v.at[0]], d0).wait()

    for t in range(NIDX - 1):
        istart(t, t)
    for t in range(NBUF - 1):
        iwait(t)
        gstart(t, t)
    plsc.subcore_barrier()

    def group(g, carry):
        for t in range(UNROLL):
            j = g * UNROLL + t
            b = t % NBUF
            gwait(b)
            sstart(t % NIDX, b)
            if t == 0:
                @pl.when(g > 0)
                def _():
                    swait(NBUF - 1)
                    dwait()
            else:
                swait((t - 1) % NBUF)
                dwait()
            dstart(t % NIDX)
            istart(jnp.minimum(j + NIDX - 1, NB - 1), (t + NIDX - 1) % NIDX)
            iwait((t + NBUF - 1) % NIDX)
            gstart((t + NBUF - 1) % NIDX, (b + NBUF - 1) % NBUF)
        return carry

    NG = (NB - (NBUF - 1) - 1) // UNROLL
    lax.fori_loop(0, NG, group, 0)

    for j in range(NG * UNROLL, NB):
        b = j % NBUF
        gwait(b)
        sstart(j % NIDX, b)
        swait((j - 1) % NBUF)
        dwait()
        dstart(j % NIDX)
        if j + NBUF - 1 <= NB - 1:
            iwait((j + NBUF - 1) % NIDX)
            gstart((j + NBUF - 1) % NIDX, (j + NBUF - 1) % NBUF)

    swait((NB - 1) % NBUF)
    dwait()
    for k in range((NIDX - 1) + NG * UNROLL - NB):
        iwait((NB + k) % NIDX)
    plsc.subcore_barrier()

    @pl.when(s < NS - 1)
    def _():
        pltpu.sync_copy(sum_sh.at[pl.ds(s * RPT, RPT)],
                        sum_out.at[c].at[pl.ds(s * RPT, RPT)])
        pltpu.sync_copy(deg_sh.at[pl.ds(s * RPT, RPT)],
                        deg_out.at[c].at[pl.ds(s * RPT, RPT)])

    @pl.when(s == NS - 1)
    def _():
        pltpu.sync_copy(sum_sh.at[pl.ds(N - RLAST, RLAST)],
                        sum_out.at[c].at[pl.ds(N - RLAST, RLAST)])
        pltpu.sync_copy(deg_sh.at[pl.ds(N - RLAST, RLAST)],
                        deg_out.at[c].at[pl.ds(N - RLAST, RLAST)])


def _sc_segment_sum(x, src_r, dst_r, zsum, zdeg, ones):
    mesh = plsc.VectorSubcoreMesh(core_axis_name="c", subcore_axis_name="s")
    fn = pl.kernel(
        _sc_body,
        out_type=[
            jax.ShapeDtypeStruct((NC, N, D), jnp.float32),
            jax.ShapeDtypeStruct((NC, N, DW), jnp.float32),
        ],
        mesh=mesh,
        scratch_types=[
            pltpu.VMEM((NIDX, B), jnp.int32),
            pltpu.VMEM((NIDX, B), jnp.int32),
            pltpu.VMEM((B, D), jnp.float32),
            pltpu.VMEM((B, D), jnp.float32),
            pltpu.VMEM((B, D), jnp.float32),
            pltpu.VMEM((B, D), jnp.float32),
            pltpu.VMEM((B, DW), jnp.float32),
            pltpu.VMEM_SHARED((N, D), jnp.float32),
            pltpu.VMEM_SHARED((N, DW), jnp.float32),
            pltpu.SemaphoreType.DMA,
            pltpu.SemaphoreType.DMA,
            pltpu.SemaphoreType.DMA,
            pltpu.SemaphoreType.DMA,
            pltpu.SemaphoreType.DMA,
            pltpu.SemaphoreType.DMA,
            pltpu.SemaphoreType.DMA,
            pltpu.SemaphoreType.DMA,
            pltpu.SemaphoreType.DMA,
            pltpu.SemaphoreType.DMA,
            pltpu.SemaphoreType.DMA,
            pltpu.SemaphoreType.DMA,
            pltpu.SemaphoreType.DMA,
            pltpu.SemaphoreType.DMA,
            pltpu.SemaphoreType.DMA,
            pltpu.SemaphoreType.DMA,
            pltpu.SemaphoreType.DMA,
        ],
        compiler_params=pltpu.CompilerParams(use_tc_tiling_on_sc=False),
        name="sc_segment_sum",
    )
    return fn(x, src_r, dst_r, zsum, zdeg, ones)


BM = 1000


def _mlp_body(x_ref, sum_ref, deg_ref, w1a_ref, w1b_ref, b1_ref, w2_ref,
              b2_ref, o_ref):
    xb = x_ref[...]
    sb = sum_ref[0] + sum_ref[1]
    dg = deg_ref[0, :, 0:1] + deg_ref[1, :, 0:1]
    mean = jnp.where(dg > 0.0, sb / jnp.maximum(dg, 1.0), xb)
    h = jnp.dot(xb, w1a_ref[...], preferred_element_type=jnp.float32)
    h += jnp.dot(mean, w1b_ref[...], preferred_element_type=jnp.float32)
    h = jnp.maximum(h + b1_ref[...], 0.0)
    o_ref[...] = (jnp.dot(h, w2_ref[...], preferred_element_type=jnp.float32)
                  + b2_ref[...])


def _mlp(x, sum_p, deg_p, w1a, w1b, b1, w2, b2):
    return pl.pallas_call(
        _mlp_body,
        grid=(N // BM,),
        in_specs=[
            pl.BlockSpec((BM, D), lambda i: (i, 0)),
            pl.BlockSpec((NC, BM, D), lambda i: (0, i, 0)),
            pl.BlockSpec((NC, BM, DW), lambda i: (0, i, 0)),
            pl.BlockSpec((D, HID), lambda i: (0, 0)),
            pl.BlockSpec((D, HID), lambda i: (0, 0)),
            pl.BlockSpec((1, HID), lambda i: (0, 0)),
            pl.BlockSpec((HID, D), lambda i: (0, 0)),
            pl.BlockSpec((1, D), lambda i: (0, 0)),
        ],
        out_specs=pl.BlockSpec((BM, D), lambda i: (i, 0)),
        out_shape=jax.ShapeDtypeStruct((N, D), jnp.float32),
        name="mlp_mixer",
    )(x, sum_p, deg_p, w1a, w1b, b1, w2, b2)


def kernel(x, edge_index, W1, b1, W2, b2):
    src_r = edge_index[0].reshape(NW, NB, B)
    dst_r = edge_index[1].reshape(NW, NB, B)
    zsum = jnp.zeros((RLAST, D), jnp.float32)
    zdeg = jnp.zeros((RLAST, DW), jnp.float32)
    ones = jnp.ones((B, DW), jnp.float32)
    sum_p, deg_p = _sc_segment_sum(x, src_r, dst_r, zsum, zdeg, ones)
    return _mlp(x, sum_p, deg_p, W1[:D], W1[D:], b1.reshape(1, HID), W2,
                b2.reshape(1, D))


# speedup vs baseline: 1.2863x; 1.0511x over previous
import jax
import jax.numpy as jnp
from jax import lax
from jax.experimental import pallas as pl
from jax.experimental.pallas import tpu as pltpu
from jax.experimental.pallas import tpu_sc as plsc

N = 10000
E = 320000
D = 128
HID = 64

NC = 2
NS = 16
NW = NC * NS
EPT = E // NW
B = 80
NB = EPT // B
NBUF = 4
NIDX = 8
UNROLL = 8
RPT = 624
RLAST = N - 15 * RPT
DW = 8


def _sc_body(x_hbm, src_hbm, dst_hbm, zsum_hbm, zdeg_hbm, ones_hbm,
             sum_out, deg_out,
             srcx_v, dstx_v, rv0, rv1, rv2, rv3, ones_v, dtmp_v, d1_v,
             sum_sh, deg_sh,
             g0, g1, g2, g3, s0, s1, s2, s3, d0,
             i0, i1, i2, i3, i4, i5, i6, i7):
    c = lax.axis_index("c")
    s = lax.axis_index("s")
    w = c * NS + s
    rvs = [rv0, rv1, rv2, rv3]
    gs = [g0, g1, g2, g3]
    ss = [s0, s1, s2, s3]
    isem = [i0, i1, i2, i3, i4, i5, i6, i7]

    @pl.when(s < NS - 1)
    def _():
        pltpu.sync_copy(zsum_hbm.at[pl.ds(0, RPT)],
                        sum_sh.at[pl.ds(s * RPT, RPT)])
        pltpu.sync_copy(zdeg_hbm.at[pl.ds(0, RPT)],
                        deg_sh.at[pl.ds(s * RPT, RPT)])

    @pl.when(s == NS - 1)
    def _():
        pltpu.sync_copy(zsum_hbm, sum_sh.at[pl.ds(N - RLAST, RLAST)])
        pltpu.sync_copy(zdeg_hbm, deg_sh.at[pl.ds(N - RLAST, RLAST)])

    pltpu.sync_copy(ones_hbm, ones_v)

    def istart(j, slot):
        pltpu.async_copy(src_hbm.at[w].at[j], srcx_v.at[slot], isem[slot])
        pltpu.async_copy(dst_hbm.at[w].at[j], dstx_v.at[slot], isem[slot])

    def iwait(slot):
        pltpu.make_async_copy(src_hbm.at[w].at[0], srcx_v.at[slot],
                              isem[slot]).wait()
        pltpu.make_async_copy(dst_hbm.at[w].at[0], dstx_v.at[slot],
                              isem[slot]).wait()

    def gstart(slot, b):
        pltpu.async_copy(x_hbm.at[srcx_v.at[slot]], rvs[b], gs[b])

    def gwait(b):
        pltpu.make_async_copy(x_hbm.at[srcx_v.at[0]], rvs[b], gs[b]).wait()

    def sstart(slot, b):
        pltpu.async_copy(rvs[b], sum_sh.at[dstx_v.at[slot]], ss[b], add=True)

    def swait(b):
        pltpu.make_async_copy(rvs[b], sum_sh.at[dstx_v.at[0]], ss[b]).wait()

    def dstart(slot):
        pltpu.async_copy(ones_v, deg_sh.at[dstx_v.at[slot]], d0, add=True)

    def dwait():
        pltpu.make_async_copy(ones_v, deg_sh.at[dstx_v.at[0]], d0).wait()

    for t in range(NIDX - 1):
        istart(t, t)
    for t in range(NBUF - 1):
        iwait(t)
        gstart(t, t)
    plsc.subcore_barrier()

    def group(g, carry):
        for t in range(UNROLL):
            j = g * UNROLL + t
            b = t % NBUF
            gwait(b)
            sstart(t % NIDX, b)
            if t == 0:
                @pl.when(g > 0)
                def _():
                    swait(NBUF - 1)
                    dwait()
            else:
                swait((t - 1) % NBUF)
                dwait()
            dstart(t % NIDX)
            istart(jnp.minimum(j + NIDX - 1, NB - 1), (t + NIDX - 1) % NIDX)
            iwait((t + NBUF - 1) % NIDX)
            gstart((t + NBUF - 1) % NIDX, (b + NBUF - 1) % NBUF)
        return carry

    NG = (NB - (NBUF - 1) - 1) // UNROLL
    lax.fori_loop(0, NG, group, 0)

    for j in range(NG * UNROLL, NB):
        b = j % NBUF
        gwait(b)
        sstart(j % NIDX, b)
        swait((j - 1) % NBUF)
        dwait()
        dstart(j % NIDX)
        if j + NBUF - 1 <= NB - 1:
            iwait((j + NBUF - 1) % NIDX)
            gstart((j + NBUF - 1) % NIDX, (j + NBUF - 1) % NBUF)

    swait((NB - 1) % NBUF)
    dwait()
    for k in range((NIDX - 1) + NG * UNROLL - NB):
        iwait((NB + k) % NIDX)
    plsc.subcore_barrier()

    iota16 = lax.iota(jnp.int32, 16)
    zero16 = jnp.zeros((16,), jnp.int32)

    def decode(row0, cnt):
        done = 0
        while done < cnt:
            chunk = min(160, cnt - done)
            pltpu.sync_copy(deg_sh.at[pl.ds(row0 + done, chunk)],
                            dtmp_v.at[pl.ds(0, chunk)])
            for k in range(chunk // 16):
                v = plsc.load_gather(dtmp_v, [k * 16 + iota16, zero16])
                d1_v[pl.ds(done + k * 16, 16)] = v
            done += chunk
        pltpu.sync_copy(d1_v.at[pl.ds(0, cnt)],
                        deg_out.at[c].at[pl.ds(row0, cnt)])

    @pl.when(s < NS - 1)
    def _():
        pltpu.sync_copy(sum_sh.at[pl.ds(s * RPT, RPT)],
                        sum_out.at[c].at[pl.ds(s * RPT, RPT)])
        decode(s * RPT, RPT)

    @pl.when(s == NS - 1)
    def _():
        pltpu.sync_copy(sum_sh.at[pl.ds(N - RLAST, RLAST)],
                        sum_out.at[c].at[pl.ds(N - RLAST, RLAST)])
        decode(N - RLAST, RLAST)


def _sc_segment_sum(x, src_r, dst_r, zsum, zdeg, ones):
    mesh = plsc.VectorSubcoreMesh(core_axis_name="c", subcore_axis_name="s")
    fn = pl.kernel(
        _sc_body,
        out_type=[
            jax.ShapeDtypeStruct((NC, N, D), jnp.float32),
            jax.ShapeDtypeStruct((NC, N), jnp.float32),
        ],
        mesh=mesh,
        scratch_types=[
            pltpu.VMEM((NIDX, B), jnp.int32),
            pltpu.VMEM((NIDX, B), jnp.int32),
            pltpu.VMEM((B, D), jnp.float32),
            pltpu.VMEM((B, D), jnp.float32),
            pltpu.VMEM((B, D), jnp.float32),
            pltpu.VMEM((B, D), jnp.float32),
            pltpu.VMEM((B, DW), jnp.float32),
            pltpu.VMEM((160, DW), jnp.float32),
            pltpu.VMEM((RLAST,), jnp.float32),
            pltpu.VMEM_SHARED((N, D), jnp.float32),
            pltpu.VMEM_SHARED((N, DW), jnp.float32),
            pltpu.SemaphoreType.DMA,
            pltpu.SemaphoreType.DMA,
            pltpu.SemaphoreType.DMA,
            pltpu.SemaphoreType.DMA,
            pltpu.SemaphoreType.DMA,
            pltpu.SemaphoreType.DMA,
            pltpu.SemaphoreType.DMA,
            pltpu.SemaphoreType.DMA,
            pltpu.SemaphoreType.DMA,
            pltpu.SemaphoreType.DMA,
            pltpu.SemaphoreType.DMA,
            pltpu.SemaphoreType.DMA,
            pltpu.SemaphoreType.DMA,
            pltpu.SemaphoreType.DMA,
            pltpu.SemaphoreType.DMA,
            pltpu.SemaphoreType.DMA,
            pltpu.SemaphoreType.DMA,
        ],
        compiler_params=pltpu.CompilerParams(use_tc_tiling_on_sc=False,
                                             needs_layout_passes=False),
        name="sc_segment_sum",
    )
    return fn(x, src_r, dst_r, zsum, zdeg, ones)


BM = 1000


def _mlp_body(x_ref, sum_ref, deg_ref, w1a_ref, w1b_ref, b1_ref, w2_ref,
              b2_ref, o_ref):
    xb = x_ref[...]
    sb = sum_ref[0] + sum_ref[1]
    dg = jnp.reshape(deg_ref[0, 0], (BM, 1))
    mean = jnp.where(dg > 0.0, sb / jnp.maximum(dg, 1.0), xb)
    h = jnp.dot(xb, w1a_ref[...], preferred_element_type=jnp.float32)
    h += jnp.dot(mean, w1b_ref[...], preferred_element_type=jnp.float32)
    h = jnp.maximum(h + b1_ref[...], 0.0)
    o_ref[...] = (jnp.dot(h, w2_ref[...], preferred_element_type=jnp.float32)
                  + b2_ref[...])


def _mlp(x, sum_p, deg_p, w1a, w1b, b1, w2, b2):
    return pl.pallas_call(
        _mlp_body,
        grid=(N // BM,),
        in_specs=[
            pl.BlockSpec((BM, D), lambda i: (i, 0)),
            pl.BlockSpec((NC, BM, D), lambda i: (0, i, 0)),
            pl.BlockSpec((1, 1, BM), lambda i: (i, 0, 0)),
            pl.BlockSpec((D, HID), lambda i: (0, 0)),
            pl.BlockSpec((D, HID), lambda i: (0, 0)),
            pl.BlockSpec((1, HID), lambda i: (0, 0)),
            pl.BlockSpec((HID, D), lambda i: (0, 0)),
            pl.BlockSpec((1, D), lambda i: (0, 0)),
        ],
        out_specs=pl.BlockSpec((BM, D), lambda i: (i, 0)),
        out_shape=jax.ShapeDtypeStruct((N, D), jnp.float32),
        name="mlp_mixer",
    )(x, sum_p, deg_p, w1a, w1b, b1, w2, b2)


def kernel(x, edge_index, W1, b1, W2, b2):
    src_r = edge_index[0].reshape(NW, NB, B)
    dst_r = edge_index[1].reshape(NW, NB, B)
    zsum = jnp.zeros((RLAST, D), jnp.float32)
    zdeg = jnp.zeros((RLAST, DW), jnp.float32)
    ones = jnp.ones((B, DW), jnp.float32)
    sum_p, deg_p = _sc_segment_sum(x, src_r, dst_r, zsum, zdeg, ones)
    dg = (deg_p[0] + deg_p[1]).reshape(N // BM, 1, BM)
    return _mlp(x, sum_p, dg, W1[:D], W1[D:], b1.reshape(1, HID), W2,
                b2.reshape(1, D))


# speedup vs baseline: 1.3219x; 1.0277x over previous
import jax
import jax.numpy as jnp
from jax import lax
from jax.experimental import pallas as pl
from jax.experimental.pallas import tpu as pltpu
from jax.experimental.pallas import tpu_sc as plsc

N = 10000
E = 320000
D = 128
HID = 64

NC = 2
NS = 16
NW = NC * NS
EPT = E // NW
B = 80
NB = EPT // B
NBUF = 4
NIDX = 8
UNROLL = 8
RPT = 624
RLAST = N - 15 * RPT
DW = 8


def _sc_body(x_hbm, src_hbm, dst_hbm, zsum_hbm, zdeg_hbm, ones_hbm,
             sum_out, deg_out,
             srcx_v, dstx_v, rv0, rv1, rv2, rv3, ones_v, dtmp_v, d1_v,
             sum_sh, deg_sh,
             g0, g1, g2, g3, s0, s1, s2, s3, d0,
             i0, i1, i2, i3, i4, i5, i6, i7):
    c = lax.axis_index("c")
    s = lax.axis_index("s")
    w = c * NS + s
    rvs = [rv0, rv1, rv2, rv3]
    gs = [g0, g1, g2, g3]
    ss = [s0, s1, s2, s3]
    isem = [i0, i1, i2, i3, i4, i5, i6, i7]

    @pl.when(s < NS - 1)
    def _():
        pltpu.sync_copy(zsum_hbm.at[pl.ds(0, RPT)],
                        sum_sh.at[pl.ds(s * RPT, RPT)])
        pltpu.sync_copy(zdeg_hbm.at[pl.ds(0, RPT)],
                        deg_sh.at[pl.ds(s * RPT, RPT)])

    @pl.when(s == NS - 1)
    def _():
        pltpu.sync_copy(zsum_hbm, sum_sh.at[pl.ds(N - RLAST, RLAST)])
        pltpu.sync_copy(zdeg_hbm, deg_sh.at[pl.ds(N - RLAST, RLAST)])

    pltpu.sync_copy(ones_hbm, ones_v)

    def istart(j, slot):
        pltpu.async_copy(src_hbm.at[w].at[j], srcx_v.at[slot], isem[slot])
        pltpu.async_copy(dst_hbm.at[w].at[j], dstx_v.at[slot], isem[slot])

    def iwait(slot):
        pltpu.make_async_copy(src_hbm.at[w].at[0], srcx_v.at[slot],
                              isem[slot]).wait()
        pltpu.make_async_copy(dst_hbm.at[w].at[0], dstx_v.at[slot],
                              isem[slot]).wait()

    def gstart(slot, b):
        pltpu.async_copy(x_hbm.at[srcx_v.at[slot]], rvs[b], gs[b])

    def gwait(b):
        pltpu.make_async_copy(x_hbm.at[srcx_v.at[0]], rvs[b], gs[b]).wait()

    def sstart(slot, b):
        pltpu.async_copy(rvs[b], sum_sh.at[dstx_v.at[slot]], ss[b], add=True)

    def swait(b):
        pltpu.make_async_copy(rvs[b], sum_sh.at[dstx_v.at[0]], ss[b]).wait()

    def dstart(slot):
        pltpu.async_copy(ones_v, deg_sh.at[dstx_v.at[slot]], d0, add=True)

    def dwait():
        pltpu.make_async_copy(ones_v, deg_sh.at[dstx_v.at[0]], d0).wait()

    for t in range(NIDX - 1):
        istart(t, t)
    for t in range(NBUF - 1):
        iwait(t)
        gstart(t, t)
    plsc.subcore_barrier()

    def group(g, carry):
        for t in range(UNROLL):
            j = g * UNROLL + t
            b = t % NBUF
            gwait(b)
            sstart(t % NIDX, b)
            if t == 0:
                @pl.when(g > 0)
                def _():
                    swait(NBUF - 1)
                    dwait()
            else:
                swait((t - 1) % NBUF)
                dwait()
            dstart(t % NIDX)
            istart(jnp.minimum(j + NIDX - 1, NB - 1), (t + NIDX - 1) % NIDX)
            iwait((t + NBUF - 1) % NIDX)
            gstart((t + NBUF - 1) % NIDX, (b + NBUF - 1) % NBUF)
        return carry

    NG = (NB - (NBUF - 1) - 1) // UNROLL
    lax.fori_loop(0, NG, group, 0)

    for j in range(NG * UNROLL, NB):
        b = j % NBUF
        gwait(b)
        sstart(j % NIDX, b)
        swait((j - 1) % NBUF)
        dwait()
        dstart(j % NIDX)
        if j + NBUF - 1 <= NB - 1:
            iwait((j + NBUF - 1) % NIDX)
            gstart((j + NBUF - 1) % NIDX, (j + NBUF - 1) % NBUF)

    swait((NB - 1) % NBUF)
    dwait()
    for k in range((NIDX - 1) + NG * UNROLL - NB):
        iwait((NB + k) % NIDX)
    plsc.subcore_barrier()

    iota16 = lax.iota(jnp.int32, 16)
    zero16 = jnp.zeros((16,), jnp.int32)

    def decode(row0, cnt):
        done = 0
        while done < cnt:
            chunk = min(160, cnt - done)
            pltpu.sync_copy(deg_sh.at[pl.ds(row0 + done, chunk)],
                            dtmp_v.at[pl.ds(0, chunk)])
            for k in range(chunk // 16):
                v = plsc.load_gather(dtmp_v, [k * 16 + iota16, zero16])
                d1_v[pl.ds(done + k * 16, 16)] = v
            done += chunk
        pltpu.sync_copy(d1_v.at[pl.ds(0, cnt)],
                        deg_out.at[c].at[pl.ds(row0, cnt)])

    @pl.when(s < NS - 1)
    def _():
        pltpu.sync_copy(sum_sh.at[pl.ds(s * RPT, RPT)],
                        sum_out.at[c].at[pl.ds(s * RPT, RPT)])
        decode(s * RPT, RPT)

    @pl.when(s == NS - 1)
    def _():
        pltpu.sync_copy(sum_sh.at[pl.ds(N - RLAST, RLAST)],
                        sum_out.at[c].at[pl.ds(N - RLAST, RLAST)])
        decode(N - RLAST, RLAST)


def _sc_segment_sum(x, src_r, dst_r, zsum, zdeg, ones):
    mesh = plsc.VectorSubcoreMesh(core_axis_name="c", subcore_axis_name="s")
    fn = pl.kernel(
        _sc_body,
        out_type=[
            jax.ShapeDtypeStruct((NC, N, D), jnp.float32),
            jax.ShapeDtypeStruct((NC, N), jnp.float32),
        ],
        mesh=mesh,
        scratch_types=[
            pltpu.VMEM((NIDX, B), jnp.int32),
            pltpu.VMEM((NIDX, B), jnp.int32),
            pltpu.VMEM((B, D), jnp.float32),
            pltpu.VMEM((B, D), jnp.float32),
            pltpu.VMEM((B, D), jnp.float32),
            pltpu.VMEM((B, D), jnp.float32),
            pltpu.VMEM((B, DW), jnp.float32),
            pltpu.VMEM((160, DW), jnp.float32),
            pltpu.VMEM((RLAST,), jnp.float32),
            pltpu.VMEM_SHARED((N, D), jnp.float32),
            pltpu.VMEM_SHARED((N, DW), jnp.float32),
            pltpu.SemaphoreType.DMA,
            pltpu.SemaphoreType.DMA,
            pltpu.SemaphoreType.DMA,
            pltpu.SemaphoreType.DMA,
            pltpu.SemaphoreType.DMA,
            pltpu.SemaphoreType.DMA,
            pltpu.SemaphoreType.DMA,
            pltpu.SemaphoreType.DMA,
            pltpu.SemaphoreType.DMA,
            pltpu.SemaphoreType.DMA,
            pltpu.SemaphoreType.DMA,
            pltpu.SemaphoreType.DMA,
            pltpu.SemaphoreType.DMA,
            pltpu.SemaphoreType.DMA,
            pltpu.SemaphoreType.DMA,
            pltpu.SemaphoreType.DMA,
            pltpu.SemaphoreType.DMA,
        ],
        compiler_params=pltpu.CompilerParams(use_tc_tiling_on_sc=False,
                                             needs_layout_passes=False),
        name="sc_segment_sum",
    )
    return fn(x, src_r, dst_r, zsum, zdeg, ones)


BM = 2000


def _mlp_body(x_ref, sum_ref, deg_ref, w1a_ref, w1b_ref, b1_ref, w2_ref,
              b2_ref, o_ref):
    xb = x_ref[...]
    sb = sum_ref[0] + sum_ref[1]
    dg = jnp.reshape(deg_ref[0, 0], (BM, 1))
    mean = jnp.where(dg > 0.0, sb / jnp.maximum(dg, 1.0), xb)
    h = jnp.dot(xb, w1a_ref[...], preferred_element_type=jnp.float32)
    h += jnp.dot(mean, w1b_ref[...], preferred_element_type=jnp.float32)
    h = jnp.maximum(h + b1_ref[...], 0.0)
    o_ref[...] = (jnp.dot(h, w2_ref[...], preferred_element_type=jnp.float32)
                  + b2_ref[...])


def _mlp(x, sum_p, deg_p, w1a, w1b, b1, w2, b2):
    return pl.pallas_call(
        _mlp_body,
        grid=(N // BM,),
        in_specs=[
            pl.BlockSpec((BM, D), lambda i: (i, 0)),
            pl.BlockSpec((NC, BM, D), lambda i: (0, i, 0)),
            pl.BlockSpec((1, 1, BM), lambda i: (i, 0, 0)),
            pl.BlockSpec((D, HID), lambda i: (0, 0)),
            pl.BlockSpec((D, HID), lambda i: (0, 0)),
            pl.BlockSpec((1, HID), lambda i: (0, 0)),
            pl.BlockSpec((HID, D), lambda i: (0, 0)),
            pl.BlockSpec((1, D), lambda i: (0, 0)),
        ],
        out_specs=pl.BlockSpec((BM, D), lambda i: (i, 0)),
        out_shape=jax.ShapeDtypeStruct((N, D), jnp.float32),
        name="mlp_mixer",
    )(x, sum_p, deg_p, w1a, w1b, b1, w2, b2)


def kernel(x, edge_index, W1, b1, W2, b2):
    src_r = edge_index[0].reshape(NW, NB, B)
    dst_r = edge_index[1].reshape(NW, NB, B)
    zsum = jnp.zeros((RLAST, D), jnp.float32)
    zdeg = jnp.zeros((RLAST, DW), jnp.float32)
    ones = jnp.ones((B, DW), jnp.float32)
    sum_p, deg_p = _sc_segment_sum(x, src_r, dst_r, zsum, zdeg, ones)
    dg = (deg_p[0] + deg_p[1]).reshape(N // BM, 1, BM)
    return _mlp(x, sum_p, dg, W1[:D], W1[D:], b1.reshape(1, HID), W2,
                b2.reshape(1, D))


# speedup vs baseline: 1.3253x; 1.0025x over previous
import jax
import jax.numpy as jnp
from jax import lax
from jax.experimental import pallas as pl
from jax.experimental.pallas import tpu as pltpu
from jax.experimental.pallas import tpu_sc as plsc

N = 10000
E = 320000
D = 128
HID = 64

NC = 2
NS = 16
NW = NC * NS
EPT = E // NW
B = 80
NB = EPT // B
NBUF = 4
NIDX = 8
UNROLL = 8
RPT = 624
RLAST = N - 15 * RPT
DW = 8


def _sc_body(x_hbm, src_hbm, dst_hbm, zsum_hbm, zdeg_hbm, ones_hbm,
             sum_out, deg_out,
             srcx_v, dstx_v, rv0, rv1, rv2, rv3, ones_v, dtmp_v, d1_v,
             sum_sh, deg_sh,
             g0, g1, g2, g3, s0, s1, s2, s3, d0,
             i0, i1, i2, i3, i4, i5, i6, i7):
    c = lax.axis_index("c")
    s = lax.axis_index("s")
    w = c * NS + s
    rvs = [rv0, rv1, rv2, rv3]
    gs = [g0, g1, g2, g3]
    ss = [s0, s1, s2, s3]
    isem = [i0, i1, i2, i3, i4, i5, i6, i7]

    @pl.when(s < NS - 1)
    def _():
        pltpu.sync_copy(zsum_hbm.at[pl.ds(0, RPT)],
                        sum_sh.at[pl.ds(s * RPT, RPT)])
        pltpu.sync_copy(zdeg_hbm.at[pl.ds(0, RPT)],
                        deg_sh.at[pl.ds(s * RPT, RPT)])

    @pl.when(s == NS - 1)
    def _():
        pltpu.sync_copy(zsum_hbm, sum_sh.at[pl.ds(N - RLAST, RLAST)])
        pltpu.sync_copy(zdeg_hbm, deg_sh.at[pl.ds(N - RLAST, RLAST)])

    pltpu.sync_copy(ones_hbm, ones_v)

    def istart(j, slot):
        pltpu.async_copy(src_hbm.at[w].at[j], srcx_v.at[slot], isem[slot])
        pltpu.async_copy(dst_hbm.at[w].at[j], dstx_v.at[slot], isem[slot])

    def iwait(slot):
        pltpu.make_async_copy(src_hbm.at[w].at[0], srcx_v.at[slot],
                              isem[slot]).wait()
        pltpu.make_async_copy(dst_hbm.at[w].at[0], dstx_v.at[slot],
                              isem[slot]).wait()

    def gstart(slot, b):
        pltpu.async_copy(x_hbm.at[srcx_v.at[slot]], rvs[b], gs[b])

    def gwait(b):
        pltpu.make_async_copy(x_hbm.at[srcx_v.at[0]], rvs[b], gs[b]).wait()

    def sstart(slot, b):
        pltpu.async_copy(rvs[b], sum_sh.at[dstx_v.at[slot]], ss[b], add=True)

    def swait(b):
        pltpu.make_async_copy(rvs[b], sum_sh.at[dstx_v.at[0]], ss[b]).wait()

    def dstart(slot):
        pltpu.async_copy(ones_v, deg_sh.at[dstx_v.at[slot]], d0, add=True)

    def dwait():
        pltpu.make_async_copy(ones_v, deg_sh.at[dstx_v.at[0]], d0).wait()

    for t in range(NIDX - 1):
        istart(t, t)
    for t in range(NBUF - 1):
        iwait(t)
        gstart(t, t)
    plsc.subcore_barrier()

    def group(g, carry):
        for t in range(UNROLL):
            j = g * UNROLL + t
            b = t % NBUF
            gwait(b)
            sstart(t % NIDX, b)
            if t == 0:
                @pl.when(g > 0)
                def _():
                    swait(NBUF - 1)
                    dwait()
            else:
                swait((t - 1) % NBUF)
                dwait()
            dstart(t % NIDX)
            istart(jnp.minimum(j + NIDX - 1, NB - 1), (t + NIDX - 1) % NIDX)
            iwait((t + NBUF - 1) % NIDX)
            gstart((t + NBUF - 1) % NIDX, (b + NBUF - 1) % NBUF)
        return carry

    NG = (NB - (NBUF - 1) - 1) // UNROLL
    lax.fori_loop(0, NG, group, 0)

    for j in range(NG * UNROLL, NB):
        b = j % NBUF
        gwait(b)
        sstart(j % NIDX, b)
        swait((j - 1) % NBUF)
        dwait()
        dstart(j % NIDX)
        if j + NBUF - 1 <= NB - 1:
            iwait((j + NBUF - 1) % NIDX)
            gstart((j + NBUF - 1) % NIDX, (j + NBUF - 1) % NBUF)

    swait((NB - 1) % NBUF)
    dwait()
    for k in range((NIDX - 1) + NG * UNROLL - NB):
        iwait((NB + k) % NIDX)
    plsc.subcore_barrier()

    iota16 = lax.iota(jnp.int32, 16)
    zero16 = jnp.zeros((16,), jnp.int32)

    def decode(row0, cnt):
        done = 0
        while done < cnt:
            chunk = min(160, cnt - done)
            pltpu.sync_copy(deg_sh.at[pl.ds(row0 + done, chunk)],
                            dtmp_v.at[pl.ds(0, chunk)])
            for k in range(chunk // 16):
                v = plsc.load_gather(dtmp_v, [k * 16 + iota16, zero16])
                d1_v[pl.ds(done + k * 16, 16)] = v
            done += chunk
        pltpu.sync_copy(d1_v.at[pl.ds(0, cnt)],
                        deg_out.at[c].at[pl.ds(row0, cnt)])

    @pl.when(s < NS - 1)
    def _():
        pltpu.sync_copy(sum_sh.at[pl.ds(s * RPT, RPT)],
                        sum_out.at[c].at[pl.ds(s * RPT, RPT)])
        decode(s * RPT, RPT)

    @pl.when(s == NS - 1)
    def _():
        pltpu.sync_copy(sum_sh.at[pl.ds(N - RLAST, RLAST)],
                        sum_out.at[c].at[pl.ds(N - RLAST, RLAST)])
        decode(N - RLAST, RLAST)


def _sc_segment_sum(x, src_r, dst_r, zsum, zdeg, ones):
    mesh = plsc.VectorSubcoreMesh(core_axis_name="c", subcore_axis_name="s")
    fn = pl.kernel(
        _sc_body,
        out_type=[
            jax.ShapeDtypeStruct((NC, N, D), jnp.float32),
            jax.ShapeDtypeStruct((NC, N), jnp.float32),
        ],
        mesh=mesh,
        scratch_types=[
            pltpu.VMEM((NIDX, B), jnp.int32),
            pltpu.VMEM((NIDX, B), jnp.int32),
            pltpu.VMEM((B, D), jnp.float32),
            pltpu.VMEM((B, D), jnp.float32),
            pltpu.VMEM((B, D), jnp.float32),
            pltpu.VMEM((B, D), jnp.float32),
            pltpu.VMEM((B, DW), jnp.float32),
            pltpu.VMEM((160, DW), jnp.float32),
            pltpu.VMEM((RLAST,), jnp.float32),
            pltpu.VMEM_SHARED((N, D), jnp.float32),
            pltpu.VMEM_SHARED((N, DW), jnp.float32),
            pltpu.SemaphoreType.DMA,
            pltpu.SemaphoreType.DMA,
            pltpu.SemaphoreType.DMA,
            pltpu.SemaphoreType.DMA,
            pltpu.SemaphoreType.DMA,
            pltpu.SemaphoreType.DMA,
            pltpu.SemaphoreType.DMA,
            pltpu.SemaphoreType.DMA,
            pltpu.SemaphoreType.DMA,
            pltpu.SemaphoreType.DMA,
            pltpu.SemaphoreType.DMA,
            pltpu.SemaphoreType.DMA,
            pltpu.SemaphoreType.DMA,
            pltpu.SemaphoreType.DMA,
            pltpu.SemaphoreType.DMA,
            pltpu.SemaphoreType.DMA,
            pltpu.SemaphoreType.DMA,
        ],
        compiler_params=pltpu.CompilerParams(use_tc_tiling_on_sc=False,
                                             needs_layout_passes=False),
        name="sc_segment_sum",
    )
    return fn(x, src_r, dst_r, zsum, zdeg, ones)


BM = 5000


def _mlp_body(x_ref, sum_ref, deg_ref, w1a_ref, w1b_ref, b1_ref, w2_ref,
              b2_ref, o_ref):
    xb = x_ref[...]
    sb = sum_ref[0] + sum_ref[1]
    dg = jnp.reshape(deg_ref[0, 0], (BM, 1))
    mean = jnp.where(dg > 0.0, sb / jnp.maximum(dg, 1.0), xb)
    h = jnp.dot(xb, w1a_ref[...], preferred_element_type=jnp.float32)
    h += jnp.dot(mean, w1b_ref[...], preferred_element_type=jnp.float32)
    h = jnp.maximum(h + b1_ref[...], 0.0)
    o_ref[...] = (jnp.dot(h, w2_ref[...], preferred_element_type=jnp.float32)
                  + b2_ref[...])


def _mlp(x, sum_p, deg_p, w1a, w1b, b1, w2, b2):
    return pl.pallas_call(
        _mlp_body,
        grid=(N // BM,),
        in_specs=[
            pl.BlockSpec((BM, D), lambda i: (i, 0)),
            pl.BlockSpec((NC, BM, D), lambda i: (0, i, 0)),
            pl.BlockSpec((1, 1, BM), lambda i: (i, 0, 0)),
            pl.BlockSpec((D, HID), lambda i: (0, 0)),
            pl.BlockSpec((D, HID), lambda i: (0, 0)),
            pl.BlockSpec((1, HID), lambda i: (0, 0)),
            pl.BlockSpec((HID, D), lambda i: (0, 0)),
            pl.BlockSpec((1, D), lambda i: (0, 0)),
        ],
        out_specs=pl.BlockSpec((BM, D), lambda i: (i, 0)),
        out_shape=jax.ShapeDtypeStruct((N, D), jnp.float32),
        name="mlp_mixer",
    )(x, sum_p, deg_p, w1a, w1b, b1, w2, b2)


def kernel(x, edge_index, W1, b1, W2, b2):
    src_r = edge_index[0].reshape(NW, NB, B)
    dst_r = edge_index[1].reshape(NW, NB, B)
    zsum = jnp.zeros((RLAST, D), jnp.float32)
    zdeg = jnp.zeros((RLAST, DW), jnp.float32)
    ones = jnp.ones((B, DW), jnp.float32)
    sum_p, deg_p = _sc_segment_sum(x, src_r, dst_r, zsum, zdeg, ones)
    dg = (deg_p[0] + deg_p[1]).reshape(N // BM, 1, BM)
    return _mlp(x, sum_p, dg, W1[:D], W1[D:], b1.reshape(1, HID), W2,
                b2.reshape(1, D))
